# Initial kernel scaffold; baseline (speedup 1.0000x reference)
import jax
import jax.numpy as jnp
from jax.experimental import pallas as pl


def kernel(query, W1, b1, W2, b2, ln_g, ln_b, Wd1, bd1, Wd2, bd2, memory_bank, memory_importance, top_k):
    raise NotImplementedError("write your pallas kernel here")


# speedup vs baseline: 1.7890x; 1.7890x over previous
import functools

import jax
import jax.numpy as jnp
from jax import lax
from jax.experimental import pallas as pl
from jax.experimental.pallas import tpu as pltpu
from jax.experimental.pallas import tpu_sc as plsc

K = 16
M_TILE = 2048
Q_BLK = 256
DEC_BLK = 2048
NEG = -3.0e38
IMAX = 2147483647


def _topk_body(m_real, m_tiles, q_ref, w1_ref, b1_ref, w2_ref, b2_ref,
               g_ref, bb_ref, bank_ref, imp_ref, vals_ref, idx_ref,
               zn_ref, bv_ref, bi_ref):
    m = pl.program_id(1)

    @pl.when(m == 0)
    def _():
        h = jax.nn.gelu(jnp.dot(q_ref[...], w1_ref[...],
                                preferred_element_type=jnp.float32) + b1_ref[...])
        z = jnp.dot(h, w2_ref[...], preferred_element_type=jnp.float32) + b2_ref[...]
        mu = jnp.mean(z, axis=-1, keepdims=True)
        var = jnp.mean((z - mu) ** 2, axis=-1, keepdims=True)
        z = (z - mu) / jnp.sqrt(var + 1e-5) * g_ref[...] + bb_ref[...]
        zn_ref[...] = z / (jnp.sqrt(jnp.sum(z * z, axis=-1, keepdims=True)) + 1e-8)
        bv_ref[...] = jnp.full((Q_BLK, K), NEG, jnp.float32)
        bi_ref[...] = jnp.zeros((Q_BLK, K), jnp.int32)

    bank = bank_ref[...]
    inv = 1.0 / (jnp.sqrt(jnp.sum(bank * bank, axis=-1, keepdims=True)) + 1e-8)
    mn = bank * inv
    s = lax.dot_general(zn_ref[...], mn, (((1,), (1,)), ((), ())),
                        preferred_element_type=jnp.float32)
    s = s * (0.5 + 0.5 * imp_ref[...])
    gcol = m * M_TILE + lax.broadcasted_iota(jnp.int32, (Q_BLK, M_TILE), 1)
    s = jnp.where(gcol < m_real, s, NEG)

    cv = jnp.concatenate([s, bv_ref[...]], axis=1)
    ci = jnp.concatenate([gcol, bi_ref[...]], axis=1)
    nv, ni = [], []
    for _ in range(K):
        mx = jnp.max(cv, axis=1, keepdims=True)
        hit = cv >= mx
        cand = jnp.min(jnp.where(hit, ci, IMAX), axis=1, keepdims=True)
        nv.append(mx)
        ni.append(cand)
        cv = jnp.where(hit & (ci == cand), NEG, cv)
    bv_ref[...] = jnp.concatenate(nv, axis=1)
    bi_ref[...] = jnp.concatenate(ni, axis=1)

    @pl.when(m == m_tiles - 1)
    def _():
        vals_ref[...] = bv_ref[...]
        idx_ref[...] = bi_ref[...]


def _run_topk(query, W1, b1, W2, b2, ln_g, ln_b, bank_p, imp_p, m_real):
    qn, f = query.shape
    m_tiles = bank_p.shape[0] // M_TILE
    body = functools.partial(_topk_body, m_real, m_tiles)
    return pl.pallas_call(
        body,
        grid=(qn // Q_BLK, m_tiles),
        in_specs=[
            pl.BlockSpec((Q_BLK, f), lambda q, m: (q, 0)),
            pl.BlockSpec(W1.shape, lambda q, m: (0, 0)),
            pl.BlockSpec(b1.shape, lambda q, m: (0, 0)),
            pl.BlockSpec(W2.shape, lambda q, m: (0, 0)),
            pl.BlockSpec(b2.shape, lambda q, m: (0, 0)),
            pl.BlockSpec(ln_g.shape, lambda q, m: (0, 0)),
            pl.BlockSpec(ln_b.shape, lambda q, m: (0, 0)),
            pl.BlockSpec((M_TILE, bank_p.shape[1]), lambda q, m: (m, 0)),
            pl.BlockSpec((1, M_TILE), lambda q, m: (0, m)),
        ],
        out_specs=[
            pl.BlockSpec((Q_BLK, K), lambda q, m: (q, 0)),
            pl.BlockSpec((Q_BLK, K), lambda q, m: (q, 0)),
        ],
        out_shape=[
            jax.ShapeDtypeStruct((qn, K), jnp.float32),
            jax.ShapeDtypeStruct((qn, K), jnp.int32),
        ],
        scratch_shapes=[
            pltpu.VMEM((Q_BLK, 64), jnp.float32),
            pltpu.VMEM((Q_BLK, K), jnp.float32),
            pltpu.VMEM((Q_BLK, K), jnp.int32),
        ],
        compiler_params=pltpu.CompilerParams(
            dimension_semantics=("arbitrary", "arbitrary")),
    )(query, W1, b1, W2, b2, ln_g, ln_b, bank_p, imp_p)


def _gather_codes(bank, flat_idx):
    b_total = flat_idx.shape[0]
    d = bank.shape[1]
    nw = 32
    b_per_w = b_total // nw
    mesh = plsc.VectorSubcoreMesh(core_axis_name="c", subcore_axis_name="s")

    @functools.partial(
        pl.kernel, mesh=mesh,
        out_type=jax.ShapeDtypeStruct((b_total, d), jnp.float32),
        compiler_params=pltpu.CompilerParams(use_tc_tiling_on_sc=False),
        scratch_types=[
            pltpu.VMEM((b_per_w,), jnp.int32),
            pltpu.VMEM((b_per_w, d), jnp.float32),
            pltpu.SemaphoreType.DMA,
        ],
    )
    def gk(table_hbm, idx_hbm, out_hbm, idx_v, rows_v, sem):
        wid = lax.axis_index("s") * 2 + lax.axis_index("c")
        base = wid * b_per_w
        pltpu.sync_copy(idx_hbm.at[pl.ds(base, b_per_w)], idx_v)
        pltpu.async_copy(table_hbm.at[idx_v], rows_v, sem).wait()
        pltpu.sync_copy(rows_v, out_hbm.at[pl.ds(base, b_per_w)])

    return gk(bank, flat_idx)


def _decode_body(codes_ref, wd1_ref, bd1_ref, wd2_ref, bd2_ref, out_ref):
    h = jax.nn.gelu(jnp.dot(codes_ref[...], wd1_ref[...],
                            preferred_element_type=jnp.float32) + bd1_ref[...])
    out_ref[...] = jnp.dot(h, wd2_ref[...],
                           preferred_element_type=jnp.float32) + bd2_ref[...]


def _run_decode(codes, Wd1, bd1, Wd2, bd2):
    b_total, d = codes.shape
    f = Wd2.shape[1]
    return pl.pallas_call(
        _decode_body,
        grid=(b_total // DEC_BLK,),
        in_specs=[
            pl.BlockSpec((DEC_BLK, d), lambda i: (i, 0)),
            pl.BlockSpec(Wd1.shape, lambda i: (0, 0)),
            pl.BlockSpec(bd1.shape, lambda i: (0, 0)),
            pl.BlockSpec(Wd2.shape, lambda i: (0, 0)),
            pl.BlockSpec(bd2.shape, lambda i: (0, 0)),
        ],
        out_specs=pl.BlockSpec((DEC_BLK, f), lambda i: (i, 0)),
        out_shape=jax.ShapeDtypeStruct((b_total, f), jnp.float32),
    )(codes, Wd1, bd1, Wd2, bd2)


def kernel(query, W1, b1, W2, b2, ln_g, ln_b, Wd1, bd1, Wd2, bd2,
           memory_bank, memory_importance, top_k):
    qn, f = query.shape
    m_real = memory_bank.shape[0]
    m_tiles = -(-m_real // M_TILE)
    m_pad = m_tiles * M_TILE
    bank_p = jnp.pad(memory_bank, ((0, m_pad - m_real), (0, 0)))
    imp_p = jnp.pad(memory_importance, (0, m_pad - m_real)).reshape(1, m_pad)
    vals, idx = _run_topk(query, W1, b1.reshape(1, -1), W2, b2.reshape(1, -1),
                          ln_g.reshape(1, -1), ln_b.reshape(1, -1),
                          bank_p, imp_p, m_real)
    codes = _gather_codes(memory_bank, idx.reshape(-1))
    decoded = _run_decode(codes, Wd1, bd1.reshape(1, -1), Wd2, bd2.reshape(1, -1))
    return decoded.reshape(qn, K, f), vals, idx
